# baseline (speedup 1.0000x reference)
import jax
import jax.numpy as jnp
from jax import lax
from jax.experimental import pallas as pl
from jax.experimental.pallas import tpu as pltpu
from jax.experimental.pallas import tpu_sc as plsc

N = 10000
D = 128
DH = 64
NUM_MSG = 10000
NC = 2
NS = 16
L = 16

HB = 160
CK = 80
MPT = 640
NHP = 10240
RT = 640


def _body(m_hbm, nodes2_hbm, zeros_hbm, out_hbm,
          sums_sh, cnts_sh,
          rows2, lidx2, ones_v, z16_v,
          sem_ldn, sem_ldr, sem_scr, sem_sco, sem_wr, sem_rd):
    core = lax.axis_index("c")
    sub = lax.axis_index("s")
    onevec = jnp.ones((L,), jnp.float32)
    cb = core * DH
    rbase = sub * RT
    mbase = sub * MPT
    bbase = sub * (MPT // CK)

    def fire_load(step, buf, rows):
        mb = mbase + step * HB
        blk = bbase + step * (HB // CK)
        dn = pltpu.async_copy(nodes2_hbm.at[pl.ds(blk, rows // CK)],
                              lidx2.at[buf, pl.ds(0, rows // CK)], sem_ldn)
        dr = pltpu.async_copy(m_hbm.at[pl.ds(mb, rows), pl.ds(cb, DH)],
                              rows2.at[buf, pl.ds(0, rows)], sem_ldr)
        return dn, dr

    ld0 = fire_load(0, 0, HB)
    ld1 = fire_load(1, 1, HB)

    for i in range(CK):
        ones_v[i, :] = onevec
    dz = pltpu.async_copy(zeros_hbm, sums_sh.at[pl.ds(rbase, RT)], sem_wr)
    dc = pltpu.async_copy(zeros_hbm.at[:, pl.ds(0, L)],
                          cnts_sh.at[pl.ds(rbase, RT)], sem_wr)
    dz.wait()
    dc.wait()
    plsc.subcore_barrier()

    def fire_scatter(buf, nchunks):
        ds = []
        for j in range(nchunks):
            ds.append(pltpu.async_copy(rows2.at[buf, pl.ds(j * CK, CK)],
                                       sums_sh.at[lidx2.at[buf, j]],
                                       sem_scr, add=True))
            ds.append(pltpu.async_copy(ones_v,
                                       cnts_sh.at[lidx2.at[buf, j]],
                                       sem_sco, add=True))
        return ds

    def wait_all(ds):
        for d in ds:
            d.wait()

    ld0[0].wait(); ld0[1].wait()
    sc0 = fire_scatter(0, HB // CK)

    ld1[0].wait(); ld1[1].wait()
    sc1 = fire_scatter(1, HB // CK)

    @pl.when(sub < NS - 1)
    def _steps_23():
        wait_all(sc0)
        ld2 = fire_load(2, 0, HB)
        ld2[0].wait(); ld2[1].wait()
        sc2 = fire_scatter(0, HB // CK)
        wait_all(sc1)
        ld3 = fire_load(3, 1, HB)
        ld3[0].wait(); ld3[1].wait()
        sc3 = fire_scatter(1, HB // CK)
        wait_all(sc2)
        wait_all(sc3)

    @pl.when(sub == NS - 1)
    def _step_2t():
        wait_all(sc0)
        ldt = fire_load(2, 0, CK)
        ldt[0].wait(); ldt[1].wait()
        sct = fire_scatter(0, 1)
        wait_all(sct)
        wait_all(sc1)

    plsc.subcore_barrier()

    dcr = pltpu.async_copy(cnts_sh.at[pl.ds(rbase, RT)], z16_v, sem_rd)

    def read_half(h, buf, rows):
        return pltpu.async_copy(sums_sh.at[pl.ds(rbase + h * HB, rows)],
                                rows2.at[buf, pl.ds(0, rows)], sem_ldr)

    def scale_half(h, buf, rows):
        def body(r4, _):
            for k in range(4):
                r = r4 * 4 + k
                s_v = z16_v[h * HB + r, :]
                for c in range(DH // L):
                    rows2[buf, r, pl.ds(c * L, L)] = \
                        rows2[buf, r, pl.ds(c * L, L)] * s_v
            return 0
        lax.fori_loop(0, rows // 4, body, 0)

    def write_half(h, buf, rows):
        return pltpu.async_copy(
            rows2.at[buf, pl.ds(0, rows)],
            out_hbm.at[pl.ds(rbase + h * HB, rows), pl.ds(cb, DH)], sem_wr)

    rd0 = read_half(0, 0, HB)
    rd1 = read_half(1, 1, HB)
    dcr.wait()

    def _recip(i, _):
        c = z16_v[i, :]
        z16_v[i, :] = jnp.where(c > 0, 1.0 / c, 0.0)
        return 0
    lax.fori_loop(0, RT, _recip, 0)

    rd0.wait()
    scale_half(0, 0, HB)
    wr0 = write_half(0, 0, HB)
    rd1.wait()
    scale_half(1, 1, HB)
    wr1 = write_half(1, 1, HB)

    @pl.when(sub < NS - 1)
    def _out_full():
        wr0.wait()
        rd2 = read_half(2, 0, HB)
        rd2.wait()
        scale_half(2, 0, HB)
        wr2 = write_half(2, 0, HB)
        wr1.wait()
        rd3 = read_half(3, 1, HB)
        rd3.wait()
        scale_half(3, 1, HB)
        wr3 = write_half(3, 1, HB)
        wr2.wait()
        wr3.wait()

    @pl.when(sub == NS - 1)
    def _out_short():
        wr0.wait()
        rdt = read_half(2, 0, CK)
        rdt.wait()
        scale_half(2, 0, CK)
        wrt = write_half(2, 0, CK)
        wrt.wait()
        wr1.wait()


_agg = pl.kernel(
    _body,
    out_type=jax.ShapeDtypeStruct((N, D), jnp.float32),
    mesh=plsc.VectorSubcoreMesh(core_axis_name="c", subcore_axis_name="s",
                                num_cores=NC, num_subcores=NS),
    compiler_params=pltpu.CompilerParams(use_tc_tiling_on_sc=False),
    scratch_types=[
        pltpu.VMEM_SHARED((NHP, DH), jnp.float32),
        pltpu.VMEM_SHARED((NHP, L), jnp.float32),
        pltpu.VMEM((2, HB, DH), jnp.float32),
        pltpu.VMEM((2, HB // CK, CK), jnp.int32),
        pltpu.VMEM((CK, L), jnp.float32),
        pltpu.VMEM((RT, L), jnp.float32),
        pltpu.SemaphoreType.DMA,
        pltpu.SemaphoreType.DMA,
        pltpu.SemaphoreType.DMA,
        pltpu.SemaphoreType.DMA,
        pltpu.SemaphoreType.DMA,
        pltpu.SemaphoreType.DMA,
    ],
)


@jax.jit
def kernel(M, nodes):
    zeros = jnp.zeros((RT, DH), jnp.float32)
    nodes2 = nodes.astype(jnp.int32).reshape(NUM_MSG // CK, CK)
    return _agg(M, nodes2, zeros)


# speedup vs baseline: 1.0390x; 1.0390x over previous
import jax
import jax.numpy as jnp
from jax import lax
from jax.experimental import pallas as pl
from jax.experimental.pallas import tpu as pltpu
from jax.experimental.pallas import tpu_sc as plsc

N = 10000
D = 128
DH = 64
NUM_MSG = 10000
NC = 2
NS = 16
L = 16

HB = 160
CK = 80
MPT = 640
NHP = 10240
RT = 640


def _body(m_hbm, nodes2_hbm, zeros_hbm, out_hbm,
          sums_sh, cnts_sh,
          rows2, lidx2, ones_v, z16_v,
          sem_ldn, sem_ldr, sem_scr, sem_sco, sem_wr, sem_rd):
    core = lax.axis_index("c")
    sub = lax.axis_index("s")
    onevec = jnp.ones((L,), jnp.float32)
    cb = core * DH
    rbase = sub * RT
    mbase = sub * MPT
    bbase = sub * (MPT // CK)

    def fire_load(step, buf, rows):
        mb = mbase + step * HB
        blk = bbase + step * (HB // CK)
        dn = pltpu.async_copy(nodes2_hbm.at[pl.ds(blk, rows // CK)],
                              lidx2.at[buf, pl.ds(0, rows // CK)], sem_ldn)
        dr = pltpu.async_copy(m_hbm.at[pl.ds(mb, rows), pl.ds(cb, DH)],
                              rows2.at[buf, pl.ds(0, rows)], sem_ldr)
        return dn, dr

    ld0 = fire_load(0, 0, HB)
    ld1 = fire_load(1, 1, HB)

    for i in range(CK):
        ones_v[i, :] = onevec
    dz = pltpu.async_copy(zeros_hbm, sums_sh.at[pl.ds(rbase, RT)], sem_wr)
    dc = pltpu.async_copy(zeros_hbm.at[:, pl.ds(0, L)],
                          cnts_sh.at[pl.ds(rbase, RT)], sem_wr)
    dz.wait()
    dc.wait()
    plsc.subcore_barrier()

    def fire_scatter(buf, nchunks):
        ds = []
        for j in range(nchunks):
            ds.append(pltpu.async_copy(rows2.at[buf, pl.ds(j * CK, CK)],
                                       sums_sh.at[lidx2.at[buf, j]],
                                       sem_scr, add=True))
            ds.append(pltpu.async_copy(ones_v,
                                       cnts_sh.at[lidx2.at[buf, j]],
                                       sem_sco, add=True))
        return ds

    def wait_all(ds):
        for d in ds:
            d.wait()

    ld0[0].wait(); ld0[1].wait()
    sc0 = fire_scatter(0, HB // CK)

    ld1[0].wait(); ld1[1].wait()
    sc1 = fire_scatter(1, HB // CK)

    @pl.when(sub < NS - 1)
    def _steps_23():
        wait_all(sc0)
        ld2 = fire_load(2, 0, HB)
        ld2[0].wait(); ld2[1].wait()
        sc2 = fire_scatter(0, HB // CK)
        wait_all(sc1)
        ld3 = fire_load(3, 1, HB)
        ld3[0].wait(); ld3[1].wait()
        sc3 = fire_scatter(1, HB // CK)
        wait_all(sc2)
        wait_all(sc3)

    @pl.when(sub == NS - 1)
    def _step_2t():
        wait_all(sc0)
        ldt = fire_load(2, 0, CK)
        ldt[0].wait(); ldt[1].wait()
        sct = fire_scatter(0, 1)
        wait_all(sct)
        wait_all(sc1)

    plsc.subcore_barrier()

    dcr = pltpu.async_copy(cnts_sh.at[pl.ds(rbase, RT)], z16_v, sem_rd)

    def read_half(h, buf, rows):
        return pltpu.async_copy(sums_sh.at[pl.ds(rbase + h * HB, rows)],
                                rows2.at[buf, pl.ds(0, rows)], sem_ldr)

    def scale_half(h, buf, rows):
        def body(r4, _):
            for k in range(4):
                r = r4 * 4 + k
                c = z16_v[h * HB + r, :]
                s_v = jnp.where(c > 0, 1.0 / c, 0.0)
                for g in range(DH // L):
                    rows2[buf, r, pl.ds(g * L, L)] = \
                        rows2[buf, r, pl.ds(g * L, L)] * s_v
            return 0
        lax.fori_loop(0, rows // 4, body, 0)

    def write_half(h, buf, rows):
        return pltpu.async_copy(
            rows2.at[buf, pl.ds(0, rows)],
            out_hbm.at[pl.ds(rbase + h * HB, rows), pl.ds(cb, DH)], sem_wr)

    rd0 = read_half(0, 0, HB)
    rd1 = read_half(1, 1, HB)
    dcr.wait()
    rd0.wait()
    scale_half(0, 0, HB)
    wr0 = write_half(0, 0, HB)
    rd1.wait()
    scale_half(1, 1, HB)
    wr1 = write_half(1, 1, HB)

    @pl.when(sub < NS - 1)
    def _out_full():
        wr0.wait()
        rd2 = read_half(2, 0, HB)
        rd2.wait()
        scale_half(2, 0, HB)
        wr2 = write_half(2, 0, HB)
        wr1.wait()
        rd3 = read_half(3, 1, HB)
        rd3.wait()
        scale_half(3, 1, HB)
        wr3 = write_half(3, 1, HB)
        wr2.wait()
        wr3.wait()

    @pl.when(sub == NS - 1)
    def _out_short():
        wr0.wait()
        rdt = read_half(2, 0, CK)
        rdt.wait()
        scale_half(2, 0, CK)
        wrt = write_half(2, 0, CK)
        wrt.wait()
        wr1.wait()


_agg = pl.kernel(
    _body,
    out_type=jax.ShapeDtypeStruct((N, D), jnp.float32),
    mesh=plsc.VectorSubcoreMesh(core_axis_name="c", subcore_axis_name="s",
                                num_cores=NC, num_subcores=NS),
    compiler_params=pltpu.CompilerParams(use_tc_tiling_on_sc=False),
    scratch_types=[
        pltpu.VMEM_SHARED((NHP, DH), jnp.float32),
        pltpu.VMEM_SHARED((NHP, L), jnp.float32),
        pltpu.VMEM((2, HB, DH), jnp.float32),
        pltpu.VMEM((2, HB // CK, CK), jnp.int32),
        pltpu.VMEM((CK, L), jnp.float32),
        pltpu.VMEM((RT, L), jnp.float32),
        pltpu.SemaphoreType.DMA,
        pltpu.SemaphoreType.DMA,
        pltpu.SemaphoreType.DMA,
        pltpu.SemaphoreType.DMA,
        pltpu.SemaphoreType.DMA,
        pltpu.SemaphoreType.DMA,
    ],
)


@jax.jit
def kernel(M, nodes):
    zeros = jnp.zeros((RT, DH), jnp.float32)
    nodes2 = nodes.astype(jnp.int32).reshape(NUM_MSG // CK, CK)
    return _agg(M, nodes2, zeros)
